# baseline (speedup 1.0000x reference)
import functools

import jax
import jax.numpy as jnp
from jax.experimental import pallas as pl
from jax.experimental.pallas import tpu as pltpu

_C_PAD = 128


def _fused_body(thr_ref, x_ref, db_ref, lab_ref, out_ref, acc_ref, *, nb, bits):
    i = pl.program_id(0)

    xb = jnp.where(x_ref[...] >= 0.0, 1.0, -1.0).astype(jnp.bfloat16)
    db = jnp.where(db_ref[...] >= 0.0, 1.0, -1.0).astype(jnp.bfloat16)

    dot = jax.lax.dot_general(
        xb, db, (((1,), (1,)), ((), ())),
        preferred_element_type=jnp.float32,
    )

    thr_dot = thr_ref[0]
    mask = (dot >= thr_dot).astype(jnp.bfloat16)

    labs = lab_ref[0]
    iota_c = jax.lax.broadcasted_iota(jnp.int32, (_C_PAD, labs.shape[1]), 0)
    oh_t = (labs == iota_c).astype(jnp.bfloat16)

    partial = jax.lax.dot_general(
        mask, oh_t, (((1,), (1,)), ((), ())),
        preferred_element_type=jnp.float32,
    )

    @pl.when(i == 0)
    def _init():
        acc_ref[...] = partial

    @pl.when(i > 0)
    def _accum():
        acc_ref[...] += partial

    @pl.when(i == nb - 1)
    def _finish():
        counts = acc_ref[...]
        sums = jnp.sum(counts, axis=1, keepdims=True)
        out_ref[...] = jnp.where(sums > 0.0, counts / jnp.maximum(sums, 1.0), 0.0)


def kernel(x, db_codes, db_labels, threshold):
    q, bits = x.shape
    n = db_codes.shape[0]

    bn = 4000 if n % 4000 == 0 else max(b for b in range(8, 4097, 8) if n % b == 0)
    nb = n // bn

    thr_dot = (jnp.asarray(bits, jnp.float32) - 2.0 * jnp.asarray(threshold, jnp.float32))
    thr_arr = thr_dot.reshape(1)

    labs3 = db_labels.reshape(nb, 1, bn)

    out = pl.pallas_call(
        functools.partial(_fused_body, nb=nb, bits=bits),
        grid=(nb,),
        in_specs=[
            pl.BlockSpec(memory_space=pltpu.SMEM),
            pl.BlockSpec((q, bits), lambda i: (0, 0)),
            pl.BlockSpec((bn, bits), lambda i: (i, 0)),
            pl.BlockSpec((1, 1, bn), lambda i: (i, 0, 0)),
        ],
        out_specs=pl.BlockSpec((q, _C_PAD), lambda i: (0, 0)),
        out_shape=jax.ShapeDtypeStruct((q, _C_PAD), jnp.float32),
        scratch_shapes=[pltpu.VMEM((q, _C_PAD), jnp.float32)],
        compiler_params=pltpu.CompilerParams(
            dimension_semantics=("arbitrary",),
        ),
    )(thr_arr, x, db_codes, labs3)

    return out[:, :100]


# speedup vs baseline: 1.5182x; 1.5182x over previous
import functools

import jax
import jax.numpy as jnp
from jax.experimental import pallas as pl
from jax.experimental.pallas import tpu as pltpu

_C_PAD = 128


def _fused_body(thr_ref, x_ref, db_ref, lab_ref, out_ref, acc_ref, *, nb, bits):
    i = pl.program_id(0)

    xb = jnp.where(x_ref[...] >= 0.0, 1.0, -1.0).astype(jnp.float8_e4m3fn)
    db = jnp.where(db_ref[...] >= 0.0, 1.0, -1.0).astype(jnp.float8_e4m3fn)

    dot = jax.lax.dot_general(
        xb, db, (((1,), (1,)), ((), ())),
        preferred_element_type=jnp.float32,
    )

    thr_dot = thr_ref[0]
    mask = (dot >= thr_dot).astype(jnp.float8_e4m3fn)

    labs = lab_ref[0]
    iota_c = jax.lax.broadcasted_iota(jnp.int32, (_C_PAD, labs.shape[1]), 0)
    oh_t = (labs == iota_c).astype(jnp.float8_e4m3fn)

    partial = jax.lax.dot_general(
        mask, oh_t, (((1,), (1,)), ((), ())),
        preferred_element_type=jnp.float32,
    )

    @pl.when(i == 0)
    def _init():
        acc_ref[...] = partial

    @pl.when(i > 0)
    def _accum():
        acc_ref[...] += partial

    @pl.when(i == nb - 1)
    def _finish():
        counts = acc_ref[...]
        sums = jnp.sum(counts, axis=1, keepdims=True)
        out_ref[...] = jnp.where(sums > 0.0, counts / jnp.maximum(sums, 1.0), 0.0)


def kernel(x, db_codes, db_labels, threshold):
    q, bits = x.shape
    n = db_codes.shape[0]

    bn = 4000 if n % 4000 == 0 else max(b for b in range(8, 4097, 8) if n % b == 0)
    nb = n // bn

    thr_dot = (jnp.asarray(bits, jnp.float32) - 2.0 * jnp.asarray(threshold, jnp.float32))
    thr_arr = thr_dot.reshape(1)

    labs3 = db_labels.reshape(nb, 1, bn)

    out = pl.pallas_call(
        functools.partial(_fused_body, nb=nb, bits=bits),
        grid=(nb,),
        in_specs=[
            pl.BlockSpec(memory_space=pltpu.SMEM),
            pl.BlockSpec((q, bits), lambda i: (0, 0)),
            pl.BlockSpec((bn, bits), lambda i: (i, 0)),
            pl.BlockSpec((1, 1, bn), lambda i: (i, 0, 0)),
        ],
        out_specs=pl.BlockSpec((q, _C_PAD), lambda i: (0, 0)),
        out_shape=jax.ShapeDtypeStruct((q, _C_PAD), jnp.float32),
        scratch_shapes=[pltpu.VMEM((q, _C_PAD), jnp.float32)],
        compiler_params=pltpu.CompilerParams(
            dimension_semantics=("arbitrary",),
        ),
    )(thr_arr, x, db_codes, labs3)

    return out[:, :100]


# speedup vs baseline: 2.3184x; 1.5271x over previous
import functools

import jax
import jax.numpy as jnp
from jax.experimental import pallas as pl
from jax.experimental.pallas import tpu as pltpu

_C_PAD = 128


def _fused_body(thr_ref, x_ref, db_ref, lab_ref, out_ref, acc_ref, *, nb):
    i = pl.program_id(0)

    xb = jnp.where(x_ref[...] >= 0.0, 1.0, -1.0).astype(jnp.float8_e4m3fn)
    db = jnp.where(db_ref[...] >= 0.0, 1.0, -1.0).astype(jnp.float8_e4m3fn)

    dot = jax.lax.dot_general(
        xb, db, (((0,), (0,)), ((), ())),
        preferred_element_type=jnp.float32,
    )

    thr_dot = thr_ref[0]
    mask = (dot >= thr_dot).astype(jnp.float8_e4m3fn)

    labs = lab_ref[0]
    iota_c = jax.lax.broadcasted_iota(jnp.int32, (_C_PAD, labs.shape[1]), 0)
    oh_t = (labs == iota_c).astype(jnp.float8_e4m3fn)

    partial = jax.lax.dot_general(
        mask, oh_t, (((1,), (1,)), ((), ())),
        preferred_element_type=jnp.float32,
    )

    @pl.when(i == 0)
    def _init():
        acc_ref[...] = partial

    @pl.when(i > 0)
    def _accum():
        acc_ref[...] += partial

    @pl.when(i == nb - 1)
    def _finish():
        counts = acc_ref[...]
        sums = jnp.sum(counts, axis=1, keepdims=True)
        out_ref[...] = jnp.where(sums > 0.0, counts / jnp.maximum(sums, 1.0), 0.0)


def kernel(x, db_codes, db_labels, threshold):
    q, bits = x.shape
    n = db_codes.shape[0]

    bn = 4096
    nb = -(-n // bn)

    x_t = x.T
    db_t = db_codes.T

    labs_pad = jnp.pad(db_labels, (0, nb * bn - n), constant_values=-1)

    thr_dot = (jnp.asarray(bits, jnp.float32) - 2.0 * jnp.asarray(threshold, jnp.float32))
    thr_arr = thr_dot.reshape(1)

    labs3 = labs_pad.reshape(nb, 1, bn)

    out = pl.pallas_call(
        functools.partial(_fused_body, nb=nb),
        grid=(nb,),
        in_specs=[
            pl.BlockSpec(memory_space=pltpu.SMEM),
            pl.BlockSpec((bits, q), lambda i: (0, 0)),
            pl.BlockSpec((bits, bn), lambda i: (0, i)),
            pl.BlockSpec((1, 1, bn), lambda i: (i, 0, 0)),
        ],
        out_specs=pl.BlockSpec((q, _C_PAD), lambda i: (0, 0)),
        out_shape=jax.ShapeDtypeStruct((q, _C_PAD), jnp.float32),
        scratch_shapes=[pltpu.VMEM((q, _C_PAD), jnp.float32)],
        compiler_params=pltpu.CompilerParams(
            dimension_semantics=("arbitrary",),
        ),
    )(thr_arr, x_t, db_t, labs3)

    return out[:, :100]
